# baseline (speedup 1.0000x reference)
import functools

import jax
import jax.numpy as jnp
from jax import lax
from jax.experimental import pallas as pl
from jax.experimental.pallas import tpu as pltpu
from jax.experimental.pallas import tpu_sc as plsc

S = 4096
D = 1024
E = 8
K = 512
H = 2048
HS = 4096
TB = 1024
NTB = S // TB

_F32 = jnp.float32
_BF16 = jnp.bfloat16


def _gelu_tanh(x):
    c = 0.7978845608028654
    return 0.5 * x * (1.0 + jnp.tanh(c * (x + 0.044715 * x * x * x)))


def _gating_body(x_ref, gwt_ref, scoresT_ref, aux_ref):
    xb = x_ref[...]
    logits = jax.lax.dot_general(
        xb.astype(_BF16), gwt_ref[...].astype(_BF16), (((1,), (0,)), ((), ())),
        preferred_element_type=_F32)
    m = jnp.max(logits, axis=1, keepdims=True)
    ex = jnp.exp(logits - m)
    sm = ex / jnp.sum(ex, axis=1, keepdims=True)
    sT = sm.T
    scoresT_ref[...] = sT

    bits = lax.bitcast_convert_type(sT, jnp.int32)

    def step(_, carry):
        lo, hi = carry
        mid = (lo + hi) // 2
        cnt = jnp.sum((bits > mid).astype(jnp.int32), axis=1, keepdims=True)
        pred = cnt >= K
        return (jnp.where(pred, mid, lo), jnp.where(pred, hi, mid))

    lo0 = jnp.zeros((E, 1), jnp.int32)
    hi0 = jnp.full((E, 1), 0x3F800001, jnp.int32)
    _, hi = lax.fori_loop(0, 31, step, (lo0, hi0))
    tstar = lax.bitcast_convert_type(hi, _F32)
    cnt_gt = jnp.sum((bits > hi).astype(jnp.int32), axis=1, keepdims=True)
    needed = (K - cnt_gt).astype(_F32)
    cols = lax.broadcasted_iota(jnp.int32, (E, 128), 1)
    aux_ref[...] = jnp.where(cols < 64, tstar, needed)


@functools.partial(jax.jit, static_argnames=())
def _gating(xf, gwt):
    return pl.pallas_call(
        _gating_body,
        in_specs=[pl.BlockSpec((S, D), lambda: (0, 0)),
                  pl.BlockSpec((D, E), lambda: (0, 0))],
        out_specs=[pl.BlockSpec((E, S), lambda: (0, 0)),
                   pl.BlockSpec((E, 128), lambda: (0, 0))],
        out_shape=[jax.ShapeDtypeStruct((E, S), _F32),
                   jax.ShapeDtypeStruct((E, 128), _F32)],
    )(xf, gwt)


NSC = 2
NSS = 16
GCH = 32


def _route_body(scoresT_hbm, aux_hbm, xf_hbm,
                idx_hbm, gat_hbm, onesT_hbm, ein_hbm,
                score_v, ones_v, idx_v, gat_v, t_v, n_v, myidx_v, rows_v,
                idx_sh, sem):
    c = lax.axis_index("c")
    sid = lax.axis_index("s")

    @pl.when(sid < 4)
    def _select():
        e = c * 4 + sid
        pltpu.sync_copy(scoresT_hbm.at[e], score_v)
        pltpu.sync_copy(aux_hbm.at[e, pl.ds(0, 16)], t_v)
        pltpu.sync_copy(aux_hbm.at[e, pl.ds(64, 16)], n_v)
        tstar = t_v[...]
        needed = n_v[...].astype(jnp.int32)

        def step(j, carry):
            off, ties = carry
            v = score_v[pl.ds(j * 16, 16)]
            gt = v > tstar
            eq = v == tstar
            eqi = jnp.where(eq, 1, 0)
            excl = plsc.cumsum(eqi) - eqi
            keep = gt | (eq & ((excl + ties) < needed))
            ones_v[pl.ds(j * 16, 16)] = jnp.where(keep, 1.0, 0.0)
            idxs = j * 16 + lax.iota(jnp.int32, 16)
            plsc.store_compressed(idx_v.at[pl.ds(off, 16)], idxs, mask=keep)
            plsc.store_compressed(gat_v.at[pl.ds(off, 16)], v, mask=keep)
            nk = jnp.sum(jnp.where(keep, 1, 0))
            ne = jnp.sum(eqi)
            return (off + nk, ties + ne)

        lax.fori_loop(0, S // 16, step, (jnp.int32(0), jnp.int32(0)),
                      unroll=2)
        pltpu.sync_copy(idx_v.at[pl.ds(0, K)], idx_hbm.at[e])
        pltpu.sync_copy(gat_v.at[pl.ds(0, K)], gat_hbm.at[e])
        pltpu.sync_copy(ones_v, onesT_hbm.at[e])
        pltpu.sync_copy(idx_v.at[pl.ds(0, K)], idx_sh.at[sid])

    plsc.subcore_barrier()

    el = sid // 4
    r0 = (sid % 4) * 128
    pltpu.sync_copy(idx_sh.at[el, pl.ds(r0, 128)], myidx_v)
    base = (c * 4 + el) * K + r0
    for t in range(128 // GCH):
        pltpu.async_copy(xf_hbm.at[myidx_v.at[pl.ds(t * GCH, GCH)]],
                         rows_v, sem).wait()
        pltpu.sync_copy(rows_v, ein_hbm.at[pl.ds(base + t * GCH, GCH)])


@functools.partial(jax.jit, static_argnames=())
def _route(scoresT, aux, xf):
    mesh = plsc.VectorSubcoreMesh(core_axis_name="c", subcore_axis_name="s",
                                  num_cores=NSC, num_subcores=NSS)
    f = pl.kernel(
        _route_body,
        out_type=[
            jax.ShapeDtypeStruct((E, K), jnp.int32),
            jax.ShapeDtypeStruct((E, K), _F32),
            jax.ShapeDtypeStruct((E, S), _F32),
            jax.ShapeDtypeStruct((E * K, D), _F32),
        ],
        mesh=mesh,
        scratch_types=[
            pltpu.VMEM((S,), _F32),
            pltpu.VMEM((S,), _F32),
            pltpu.VMEM((K + 32,), jnp.int32),
            pltpu.VMEM((K + 32,), _F32),
            pltpu.VMEM((16,), _F32),
            pltpu.VMEM((16,), _F32),
            pltpu.VMEM((128,), jnp.int32),
            pltpu.VMEM((GCH, D), _F32),
            pltpu.VMEM_SHARED((4, K), jnp.int32),
            pltpu.SemaphoreType.DMA,
        ],
        compiler_params=pltpu.CompilerParams(needs_layout_passes=False),
    )
    return f(scoresT, aux, xf)


def _dense_body(x_ref, cpw1_ref, cpb1_ref, cpw2_ref, cpb2_ref,
                sw1_ref, sb1_ref, sw2_ref, sb2_ref,
                cp_ref, shared_ref):
    xb_bf = x_ref[...].astype(_BF16)

    hcp = jax.lax.dot_general(
        xb_bf, cpw1_ref[...].astype(_BF16), (((1,), (0,)), ((), ())),
        preferred_element_type=_F32) + cpb1_ref[...]
    hcp = hcp * jax.nn.sigmoid(hcp)
    cp = jax.lax.dot_general(
        hcp.astype(_BF16), cpw2_ref[...].astype(_BF16), (((1,), (0,)), ((), ())),
        preferred_element_type=_F32) + cpb2_ref[...]
    cp_ref[...] = cp

    hs = jax.lax.dot_general(
        xb_bf, sw1_ref[...].astype(_BF16), (((1,), (0,)), ((), ())),
        preferred_element_type=_F32) + sb1_ref[...]
    hs = _gelu_tanh(hs)
    sh = jax.lax.dot_general(
        hs.astype(_BF16), sw2_ref[...].astype(_BF16), (((1,), (0,)), ((), ())),
        preferred_element_type=_F32) + sb2_ref[...]
    shared_ref[...] = sh


@functools.partial(jax.jit, static_argnames=())
def _dense(xf, cp_w1, cp_b1, cp_w2, cp_b2, sw1, sb1, sw2, sb2):
    full = lambda shape: pl.BlockSpec(shape, lambda i: (0,) * len(shape))
    return pl.pallas_call(
        _dense_body,
        grid=(NTB,),
        in_specs=[
            pl.BlockSpec((TB, D), lambda i: (i, 0)),
            full((D, D)),
            full((1, D)),
            full((D, E)),
            full((1, E)),
            full((D, HS)),
            full((1, HS)),
            full((HS, D)),
            full((1, D)),
        ],
        out_specs=[
            pl.BlockSpec((TB, E), lambda i: (i, 0)),
            pl.BlockSpec((TB, D), lambda i: (i, 0)),
        ],
        out_shape=[
            jax.ShapeDtypeStruct((S, E), _F32),
            jax.ShapeDtypeStruct((S, D), _F32),
        ],
        compiler_params=pltpu.CompilerParams(
            vmem_limit_bytes=65 * 1024 * 1024),
    )(xf, cp_w1, cp_b1, cp_w2, cp_b2, sw1, sb1, sw2, sb2)


H2 = H // 2


def _moe_combine_body(ein_ref, w1_ref, b1_ref, w2_ref, b2_ref, gat_ref,
                      idx_ref, out_ref, gacc):
    e = pl.program_id(0)
    h = pl.program_id(1)

    @pl.when((e == 0) & (h == 0))
    def _init():
        out_ref[...] = jnp.zeros((S, D), _F32)

    ein = ein_ref[0].astype(_BF16)
    hid = jax.lax.dot_general(
        ein, w1_ref[0].astype(_BF16), (((1,), (0,)), ((), ())),
        preferred_element_type=_F32) + b1_ref[0]
    hid = _gelu_tanh(hid)
    part = jax.lax.dot_general(
        hid.astype(_BF16), w2_ref[0].astype(_BF16), (((1,), (0,)), ((), ())),
        preferred_element_type=_F32)

    @pl.when(h == 0)
    def _acc0():
        gacc[...] = part + b2_ref[0]

    @pl.when(h == 1)
    def _acc1():
        gated = ((gacc[...] + part)
                 * gat_ref[0].reshape(K, 1)).astype(_BF16)
        idxv = idx_ref[0].astype(jnp.int16)
        tok = lax.broadcasted_iota(jnp.int16, (S, K), 0)
        oh = (tok == idxv).astype(_BF16)
        out_ref[...] += jax.lax.dot_general(
            oh, gated, (((1,), (0,)), ((), ())), preferred_element_type=_F32)


@functools.partial(jax.jit, static_argnames=())
def _moe_combine(ein, W1, b1, W2, b2, gating, idx):
    return pl.pallas_call(
        _moe_combine_body,
        grid=(E, 2),
        in_specs=[
            pl.BlockSpec((1, K, D), lambda e, h: (e, 0, 0)),
            pl.BlockSpec((1, D, H2), lambda e, h: (e, 0, h)),
            pl.BlockSpec((1, 1, H2), lambda e, h: (e, 0, h)),
            pl.BlockSpec((1, H2, D), lambda e, h: (e, h, 0)),
            pl.BlockSpec((1, 1, D), lambda e, h: (e, 0, 0)),
            pl.BlockSpec((1, 1, K), lambda e, h: (e, 0, 0)),
            pl.BlockSpec((1, 1, K), lambda e, h: (e, 0, 0)),
        ],
        out_specs=pl.BlockSpec((S, D), lambda e, h: (0, 0)),
        out_shape=jax.ShapeDtypeStruct((S, D), _F32),
        scratch_shapes=[pltpu.VMEM((K, D), _F32)],
        compiler_params=pltpu.CompilerParams(
            vmem_limit_bytes=65 * 1024 * 1024),
    )(ein, W1, b1, W2, b2, gating, idx)


def kernel(x, gate_weight, W1, b1, W2, b2, cp_w1, cp_b1, cp_w2, cp_b2,
           sw1, sb1, sw2, sb2):
    Bx, sx, Dx = x.shape
    xf = x.reshape(S, D)

    scoresT, aux = _gating(xf, gate_weight.T)
    cp, shared = _dense(xf, cp_w1, cp_b1.reshape(1, D), cp_w2,
                        cp_b2.reshape(1, E), sw1, sb1.reshape(1, HS), sw2,
                        sb2.reshape(1, D))
    index, gating, onesT, ein = _route(scoresT, aux, xf)

    y = _moe_combine(ein.reshape(E, K, D), W1, b1.reshape(E, 1, H),
                     W2, b2.reshape(E, 1, D), gating.reshape(E, 1, K),
                     index.reshape(E, 1, K))
    x_out = (y + shared).reshape(Bx, sx, Dx)
    ones = onesT.T.reshape(Bx, sx, E)
    cp_out = cp.reshape(Bx, sx, E)
    return (x_out, ones, cp_out)


# speedup vs baseline: 1.0344x; 1.0344x over previous
import functools

import jax
import jax.numpy as jnp
from jax import lax
from jax.experimental import pallas as pl
from jax.experimental.pallas import tpu as pltpu
from jax.experimental.pallas import tpu_sc as plsc

S = 4096
D = 1024
E = 8
K = 512
H = 2048
HS = 4096
TB = 512
NTB = S // TB

_F32 = jnp.float32
_BF16 = jnp.bfloat16


def _gelu_tanh(x):
    c = 0.7978845608028654
    return 0.5 * x * (1.0 + jnp.tanh(c * (x + 0.044715 * x * x * x)))


def _gating_body(x_ref, gwt_ref, scoresT_ref, aux_ref):
    xb = x_ref[...]
    logits = jax.lax.dot_general(
        xb.astype(_BF16), gwt_ref[...].astype(_BF16), (((1,), (0,)), ((), ())),
        preferred_element_type=_F32)
    m = jnp.max(logits, axis=1, keepdims=True)
    ex = jnp.exp(logits - m)
    sm = ex / jnp.sum(ex, axis=1, keepdims=True)
    sT = sm.T
    scoresT_ref[...] = sT

    bits = lax.bitcast_convert_type(sT, jnp.int32)

    def step(_, carry):
        lo, hi = carry
        mid = (lo + hi) // 2
        cnt = jnp.sum((bits > mid).astype(jnp.int32), axis=1, keepdims=True)
        pred = cnt >= K
        return (jnp.where(pred, mid, lo), jnp.where(pred, hi, mid))

    lo0 = jnp.zeros((E, 1), jnp.int32)
    hi0 = jnp.full((E, 1), 0x3F800001, jnp.int32)
    _, hi = lax.fori_loop(0, 31, step, (lo0, hi0))
    tstar = lax.bitcast_convert_type(hi, _F32)
    cnt_gt = jnp.sum((bits > hi).astype(jnp.int32), axis=1, keepdims=True)
    needed = (K - cnt_gt).astype(_F32)
    cols = lax.broadcasted_iota(jnp.int32, (E, 128), 1)
    aux_ref[...] = jnp.where(cols < 64, tstar, needed)


@functools.partial(jax.jit, static_argnames=())
def _gating(xf, gwt):
    return pl.pallas_call(
        _gating_body,
        in_specs=[pl.BlockSpec((S, D), lambda: (0, 0)),
                  pl.BlockSpec((D, E), lambda: (0, 0))],
        out_specs=[pl.BlockSpec((E, S), lambda: (0, 0)),
                   pl.BlockSpec((E, 128), lambda: (0, 0))],
        out_shape=[jax.ShapeDtypeStruct((E, S), _F32),
                   jax.ShapeDtypeStruct((E, 128), _F32)],
    )(xf, gwt)


NSC = 2
NSS = 16
GCH = 32


def _route_body(scoresT_hbm, aux_hbm, xf_hbm,
                idx_hbm, gat_hbm, onesT_hbm, ein_hbm,
                score_v, ones_v, idx_v, gat_v, t_v, n_v, myidx_v, rows_v,
                idx_sh, sem):
    c = lax.axis_index("c")
    sid = lax.axis_index("s")

    @pl.when(sid < 4)
    def _select():
        e = c * 4 + sid
        pltpu.sync_copy(scoresT_hbm.at[e], score_v)
        pltpu.sync_copy(aux_hbm.at[e, pl.ds(0, 16)], t_v)
        pltpu.sync_copy(aux_hbm.at[e, pl.ds(64, 16)], n_v)
        tstar = t_v[...]
        needed = n_v[...].astype(jnp.int32)

        def step(j, carry):
            off, ties = carry
            v = score_v[pl.ds(j * 16, 16)]
            gt = v > tstar
            eq = v == tstar
            eqi = jnp.where(eq, 1, 0)
            excl = plsc.cumsum(eqi) - eqi
            keep = gt | (eq & ((excl + ties) < needed))
            ones_v[pl.ds(j * 16, 16)] = jnp.where(keep, 1.0, 0.0)
            idxs = j * 16 + lax.iota(jnp.int32, 16)
            plsc.store_compressed(idx_v.at[pl.ds(off, 16)], idxs, mask=keep)
            plsc.store_compressed(gat_v.at[pl.ds(off, 16)], v, mask=keep)
            nk = jnp.sum(jnp.where(keep, 1, 0))
            ne = jnp.sum(eqi)
            return (off + nk, ties + ne)

        lax.fori_loop(0, S // 16, step, (jnp.int32(0), jnp.int32(0)),
                      unroll=2)
        pltpu.sync_copy(idx_v.at[pl.ds(0, K)], idx_hbm.at[e])
        pltpu.sync_copy(gat_v.at[pl.ds(0, K)], gat_hbm.at[e])
        pltpu.sync_copy(ones_v, onesT_hbm.at[e])
        pltpu.sync_copy(idx_v.at[pl.ds(0, K)], idx_sh.at[sid])

    plsc.subcore_barrier()

    el = sid // 4
    r0 = (sid % 4) * 128
    pltpu.sync_copy(idx_sh.at[el, pl.ds(r0, 128)], myidx_v)
    base = (c * 4 + el) * K + r0
    for t in range(128 // GCH):
        pltpu.async_copy(xf_hbm.at[myidx_v.at[pl.ds(t * GCH, GCH)]],
                         rows_v, sem).wait()
        pltpu.sync_copy(rows_v, ein_hbm.at[pl.ds(base + t * GCH, GCH)])


@functools.partial(jax.jit, static_argnames=())
def _route(scoresT, aux, xf):
    mesh = plsc.VectorSubcoreMesh(core_axis_name="c", subcore_axis_name="s",
                                  num_cores=NSC, num_subcores=NSS)
    f = pl.kernel(
        _route_body,
        out_type=[
            jax.ShapeDtypeStruct((E, K), jnp.int32),
            jax.ShapeDtypeStruct((E, K), _F32),
            jax.ShapeDtypeStruct((E, S), _F32),
            jax.ShapeDtypeStruct((E * K, D), _F32),
        ],
        mesh=mesh,
        scratch_types=[
            pltpu.VMEM((S,), _F32),
            pltpu.VMEM((S,), _F32),
            pltpu.VMEM((K + 32,), jnp.int32),
            pltpu.VMEM((K + 32,), _F32),
            pltpu.VMEM((16,), _F32),
            pltpu.VMEM((16,), _F32),
            pltpu.VMEM((128,), jnp.int32),
            pltpu.VMEM((GCH, D), _F32),
            pltpu.VMEM_SHARED((4, K), jnp.int32),
            pltpu.SemaphoreType.DMA,
        ],
        compiler_params=pltpu.CompilerParams(needs_layout_passes=False),
    )
    return f(scoresT, aux, xf)


def _dense_body(x_ref, cpw1_ref, cpb1_ref, cpw2_ref, cpb2_ref,
                sw1_ref, sb1_ref, sw2_ref, sb2_ref,
                cp_ref, shared_ref):
    xb_bf = x_ref[...].astype(_BF16)

    hcp = jax.lax.dot_general(
        xb_bf, cpw1_ref[...].astype(_BF16), (((1,), (0,)), ((), ())),
        preferred_element_type=_F32) + cpb1_ref[...]
    hcp = hcp * jax.nn.sigmoid(hcp)
    cp = jax.lax.dot_general(
        hcp.astype(_BF16), cpw2_ref[...].astype(_BF16), (((1,), (0,)), ((), ())),
        preferred_element_type=_F32) + cpb2_ref[...]
    cp_ref[...] = cp

    hs = jax.lax.dot_general(
        xb_bf, sw1_ref[...].astype(_BF16), (((1,), (0,)), ((), ())),
        preferred_element_type=_F32) + sb1_ref[...]
    hs = _gelu_tanh(hs)
    sh = jax.lax.dot_general(
        hs.astype(_BF16), sw2_ref[...].astype(_BF16), (((1,), (0,)), ((), ())),
        preferred_element_type=_F32) + sb2_ref[...]
    shared_ref[...] = sh


@functools.partial(jax.jit, static_argnames=())
def _dense(xf, cp_w1, cp_b1, cp_w2, cp_b2, sw1, sb1, sw2, sb2):
    full = lambda shape: pl.BlockSpec(shape, lambda i: (0,) * len(shape))
    return pl.pallas_call(
        _dense_body,
        grid=(NTB,),
        in_specs=[
            pl.BlockSpec((TB, D), lambda i: (i, 0)),
            full((D, D)),
            full((1, D)),
            full((D, E)),
            full((1, E)),
            full((D, HS)),
            full((1, HS)),
            full((HS, D)),
            full((1, D)),
        ],
        out_specs=[
            pl.BlockSpec((TB, E), lambda i: (i, 0)),
            pl.BlockSpec((TB, D), lambda i: (i, 0)),
        ],
        out_shape=[
            jax.ShapeDtypeStruct((S, E), _F32),
            jax.ShapeDtypeStruct((S, D), _F32),
        ],
        compiler_params=pltpu.CompilerParams(
            vmem_limit_bytes=65 * 1024 * 1024),
    )(xf, cp_w1, cp_b1, cp_w2, cp_b2, sw1, sb1, sw2, sb2)


def _moe_combine_body(ein_ref, w1_ref, b1_ref, w2_ref, b2_ref, gat_ref,
                      idx_ref, out_ref):
    e = pl.program_id(0)

    @pl.when(e == 0)
    def _init():
        out_ref[...] = jnp.zeros((S, D), _F32)

    ein = ein_ref[0].astype(_BF16)
    hid = jax.lax.dot_general(
        ein, w1_ref[0].astype(_BF16), (((1,), (0,)), ((), ())),
        preferred_element_type=_F32) + b1_ref[0]
    hid = _gelu_tanh(hid)
    out = jax.lax.dot_general(
        hid.astype(_BF16), w2_ref[0].astype(_BF16), (((1,), (0,)), ((), ())),
        preferred_element_type=_F32) + b2_ref[0]
    gated = (out * gat_ref[0].reshape(K, 1)).astype(_BF16)

    idxv = idx_ref[0].astype(jnp.int16)
    tok = lax.broadcasted_iota(jnp.int16, (S, K), 0)
    oh = (tok == idxv).astype(_BF16)
    out_ref[...] += jax.lax.dot_general(
        oh, gated, (((1,), (0,)), ((), ())), preferred_element_type=_F32)


@functools.partial(jax.jit, static_argnames=())
def _moe_combine(ein, W1, b1, W2, b2, gating, idx):
    return pl.pallas_call(
        _moe_combine_body,
        grid=(E,),
        in_specs=[
            pl.BlockSpec((1, K, D), lambda e: (e, 0, 0)),
            pl.BlockSpec((1, D, H), lambda e: (e, 0, 0)),
            pl.BlockSpec((1, 1, H), lambda e: (e, 0, 0)),
            pl.BlockSpec((1, H, D), lambda e: (e, 0, 0)),
            pl.BlockSpec((1, 1, D), lambda e: (e, 0, 0)),
            pl.BlockSpec((1, 1, K), lambda e: (e, 0, 0)),
            pl.BlockSpec((1, 1, K), lambda e: (e, 0, 0)),
        ],
        out_specs=pl.BlockSpec((S, D), lambda e: (0, 0)),
        out_shape=jax.ShapeDtypeStruct((S, D), _F32),
        compiler_params=pltpu.CompilerParams(
            vmem_limit_bytes=65 * 1024 * 1024),
    )(ein, W1, b1, W2, b2, gating, idx)


def kernel(x, gate_weight, W1, b1, W2, b2, cp_w1, cp_b1, cp_w2, cp_b2,
           sw1, sb1, sw2, sb2):
    Bx, sx, Dx = x.shape
    xf = x.reshape(S, D)

    scoresT, aux = _gating(xf, gate_weight.T)
    cp, shared = _dense(xf, cp_w1, cp_b1.reshape(1, D), cp_w2,
                        cp_b2.reshape(1, E), sw1, sb1.reshape(1, HS), sw2,
                        sb2.reshape(1, D))
    index, gating, onesT, ein = _route(scoresT, aux, xf)

    y = _moe_combine(ein.reshape(E, K, D), W1, b1.reshape(E, 1, H),
                     W2, b2.reshape(E, 1, D), gating.reshape(E, 1, K),
                     index.reshape(E, 1, K))
    x_out = (y + shared).reshape(Bx, sx, Dx)
    ones = onesT.T.reshape(Bx, sx, E)
    cp_out = cp.reshape(Bx, sx, E)
    return (x_out, ones, cp_out)


# speedup vs baseline: 1.0870x; 1.0508x over previous
import functools

import jax
import jax.numpy as jnp
from jax import lax
from jax.experimental import pallas as pl
from jax.experimental.pallas import tpu as pltpu
from jax.experimental.pallas import tpu_sc as plsc

S = 4096
D = 1024
E = 8
K = 512
H = 2048
HS = 4096
TB = 512
NTB = S // TB

_F32 = jnp.float32
_BF16 = jnp.bfloat16


def _gelu_tanh(x):
    c = 0.7978845608028654
    return 0.5 * x * (1.0 + jnp.tanh(c * (x + 0.044715 * x * x * x)))


def _gating_body(x_ref, gwt_ref, scoresT_ref, aux_ref):
    xb = x_ref[...]
    logits = jax.lax.dot_general(
        xb.astype(_BF16), gwt_ref[...].astype(_BF16), (((1,), (0,)), ((), ())),
        preferred_element_type=_F32)
    m = jnp.max(logits, axis=1, keepdims=True)
    ex = jnp.exp(logits - m)
    sm = ex / jnp.sum(ex, axis=1, keepdims=True)
    sT = sm.T
    scoresT_ref[...] = sT

    bits = lax.bitcast_convert_type(sT, jnp.int32)

    def step(_, carry):
        lo, hi = carry
        mid = (lo + hi) // 2
        cnt = jnp.sum((bits > mid).astype(jnp.int32), axis=1, keepdims=True)
        pred = cnt >= K
        return (jnp.where(pred, mid, lo), jnp.where(pred, hi, mid))

    lo0 = jnp.zeros((E, 1), jnp.int32)
    hi0 = jnp.full((E, 1), 0x3F800001, jnp.int32)
    _, hi = lax.fori_loop(0, 31, step, (lo0, hi0))
    tstar = lax.bitcast_convert_type(hi, _F32)
    cnt_gt = jnp.sum((bits > hi).astype(jnp.int32), axis=1, keepdims=True)
    needed = (K - cnt_gt).astype(_F32)
    cols = lax.broadcasted_iota(jnp.int32, (E, 128), 1)
    aux_ref[...] = jnp.where(cols < 64, tstar, needed)


@functools.partial(jax.jit, static_argnames=())
def _gating(xf, gwt):
    return pl.pallas_call(
        _gating_body,
        in_specs=[pl.BlockSpec((S, D), lambda: (0, 0)),
                  pl.BlockSpec((D, E), lambda: (0, 0))],
        out_specs=[pl.BlockSpec((E, S), lambda: (0, 0)),
                   pl.BlockSpec((E, 128), lambda: (0, 0))],
        out_shape=[jax.ShapeDtypeStruct((E, S), _F32),
                   jax.ShapeDtypeStruct((E, 128), _F32)],
    )(xf, gwt)


NSC = 2
NSS = 16
GCH = 32


def _route_body(scoresT_hbm, aux_hbm, xf_hbm,
                idx_hbm, gat_hbm, onesT_hbm, ein_hbm,
                score_v, ones_v, idx_v, gat_v, t_v, n_v, myidx_v, rows_v,
                idx_sh, sem):
    c = lax.axis_index("c")
    sid = lax.axis_index("s")

    @pl.when(sid < 4)
    def _select():
        e = c * 4 + sid
        pltpu.sync_copy(scoresT_hbm.at[e], score_v)
        pltpu.sync_copy(aux_hbm.at[e, pl.ds(0, 16)], t_v)
        pltpu.sync_copy(aux_hbm.at[e, pl.ds(64, 16)], n_v)
        tstar = t_v[...]
        needed = n_v[...].astype(jnp.int32)

        def step(j, carry):
            off, ties = carry
            v = score_v[pl.ds(j * 16, 16)]
            gt = v > tstar
            eq = v == tstar
            eqi = jnp.where(eq, 1, 0)
            excl = plsc.cumsum(eqi) - eqi
            keep = gt | (eq & ((excl + ties) < needed))
            ones_v[pl.ds(j * 16, 16)] = jnp.where(keep, 1.0, 0.0)
            idxs = j * 16 + lax.iota(jnp.int32, 16)
            plsc.store_compressed(idx_v.at[pl.ds(off, 16)], idxs, mask=keep)
            plsc.store_compressed(gat_v.at[pl.ds(off, 16)], v, mask=keep)
            nk = jnp.sum(jnp.where(keep, 1, 0))
            ne = jnp.sum(eqi)
            return (off + nk, ties + ne)

        lax.fori_loop(0, S // 16, step, (jnp.int32(0), jnp.int32(0)),
                      unroll=2)
        pltpu.sync_copy(idx_v.at[pl.ds(0, K)], idx_hbm.at[e])
        pltpu.sync_copy(gat_v.at[pl.ds(0, K)], gat_hbm.at[e])
        pltpu.sync_copy(ones_v, onesT_hbm.at[e])
        pltpu.sync_copy(idx_v.at[pl.ds(0, K)], idx_sh.at[sid])

    plsc.subcore_barrier()

    el = sid // 4
    r0 = (sid % 4) * 128
    pltpu.sync_copy(idx_sh.at[el, pl.ds(r0, 128)], myidx_v)
    base = (c * 4 + el) * K + r0
    for t in range(128 // GCH):
        pltpu.async_copy(xf_hbm.at[myidx_v.at[pl.ds(t * GCH, GCH)]],
                         rows_v, sem).wait()
        pltpu.sync_copy(rows_v, ein_hbm.at[pl.ds(base + t * GCH, GCH)])


@functools.partial(jax.jit, static_argnames=())
def _route(scoresT, aux, xf):
    mesh = plsc.VectorSubcoreMesh(core_axis_name="c", subcore_axis_name="s",
                                  num_cores=NSC, num_subcores=NSS)
    f = pl.kernel(
        _route_body,
        out_type=[
            jax.ShapeDtypeStruct((E, K), jnp.int32),
            jax.ShapeDtypeStruct((E, K), _F32),
            jax.ShapeDtypeStruct((E, S), _F32),
            jax.ShapeDtypeStruct((E * K, D), _F32),
        ],
        mesh=mesh,
        scratch_types=[
            pltpu.VMEM((S,), _F32),
            pltpu.VMEM((S,), _F32),
            pltpu.VMEM((K + 32,), jnp.int32),
            pltpu.VMEM((K + 32,), _F32),
            pltpu.VMEM((16,), _F32),
            pltpu.VMEM((16,), _F32),
            pltpu.VMEM((128,), jnp.int32),
            pltpu.VMEM((GCH, D), _F32),
            pltpu.VMEM_SHARED((4, K), jnp.int32),
            pltpu.SemaphoreType.DMA,
        ],
        compiler_params=pltpu.CompilerParams(needs_layout_passes=False),
    )
    return f(scoresT, aux, xf)


def _dense_body(x_ref, cpw1_ref, cpb1_ref, cpw2_ref, cpb2_ref,
                sw1_ref, sb1_ref, sw2_ref, sb2_ref,
                cp_ref, shared_ref):
    xb_bf = x_ref[...].astype(_BF16)

    hcp = jax.lax.dot_general(
        xb_bf, cpw1_ref[...].astype(_BF16), (((1,), (0,)), ((), ())),
        preferred_element_type=_F32) + cpb1_ref[...]
    hcp = hcp * jax.nn.sigmoid(hcp)
    cp = jax.lax.dot_general(
        hcp.astype(_BF16), cpw2_ref[...].astype(_BF16), (((1,), (0,)), ((), ())),
        preferred_element_type=_F32) + cpb2_ref[...]
    cp_ref[...] = cp

    hs = jax.lax.dot_general(
        xb_bf, sw1_ref[...].astype(_BF16), (((1,), (0,)), ((), ())),
        preferred_element_type=_F32) + sb1_ref[...]
    hs = _gelu_tanh(hs)
    sh = jax.lax.dot_general(
        hs.astype(_BF16), sw2_ref[...].astype(_BF16), (((1,), (0,)), ((), ())),
        preferred_element_type=_F32) + sb2_ref[...]
    shared_ref[...] = sh


@functools.partial(jax.jit, static_argnames=())
def _dense(xf, cp_w1, cp_b1, cp_w2, cp_b2, sw1, sb1, sw2, sb2):
    full = lambda shape: pl.BlockSpec(shape, lambda i: (0,) * len(shape))
    return pl.pallas_call(
        _dense_body,
        grid=(NTB,),
        in_specs=[
            pl.BlockSpec((TB, D), lambda i: (i, 0)),
            full((D, D)),
            full((1, D)),
            full((D, E)),
            full((1, E)),
            full((D, HS)),
            full((1, HS)),
            full((HS, D)),
            full((1, D)),
        ],
        out_specs=[
            pl.BlockSpec((TB, E), lambda i: (i, 0)),
            pl.BlockSpec((TB, D), lambda i: (i, 0)),
        ],
        out_shape=[
            jax.ShapeDtypeStruct((S, E), _F32),
            jax.ShapeDtypeStruct((S, D), _F32),
        ],
        compiler_params=pltpu.CompilerParams(
            vmem_limit_bytes=65 * 1024 * 1024),
    )(xf, cp_w1, cp_b1, cp_w2, cp_b2, sw1, sb1, sw2, sb2)


def _moe_combine_body(ein_ref, w1_ref, b1_ref, w2_ref, b2_ref, gat_ref,
                      idx_ref, shared_ref, out_ref, sem):
    e = pl.program_id(0)
    shcopy = pltpu.make_async_copy(shared_ref, out_ref, sem)

    @pl.when(e == 0)
    def _init():
        shcopy.start()

    ein = ein_ref[0].astype(_BF16)
    hid = jax.lax.dot_general(
        ein, w1_ref[0].astype(_BF16), (((1,), (0,)), ((), ())),
        preferred_element_type=_F32) + b1_ref[0]
    hid = _gelu_tanh(hid)
    out = jax.lax.dot_general(
        hid.astype(_BF16), w2_ref[0].astype(_BF16), (((1,), (0,)), ((), ())),
        preferred_element_type=_F32) + b2_ref[0]
    gated = (out * gat_ref[0].reshape(K, 1)).astype(_BF16)

    @pl.when(e == 0)
    def _wait():
        shcopy.wait()

    idxv = idx_ref[0].astype(jnp.int16)
    for half in range(2):
        tok = (half * (S // 2)
               + lax.broadcasted_iota(jnp.int16, (S // 2, K), 0))
        oh = (tok == idxv).astype(_BF16)
        out_ref[pl.ds(half * (S // 2), S // 2), :] += jax.lax.dot_general(
            oh, gated, (((1,), (0,)), ((), ())), preferred_element_type=_F32)


@functools.partial(jax.jit, static_argnames=())
def _moe_combine(ein, W1, b1, W2, b2, gating, idx, shared):
    return pl.pallas_call(
        _moe_combine_body,
        grid=(E,),
        in_specs=[
            pl.BlockSpec((1, K, D), lambda e: (e, 0, 0)),
            pl.BlockSpec((1, D, H), lambda e: (e, 0, 0)),
            pl.BlockSpec((1, 1, H), lambda e: (e, 0, 0)),
            pl.BlockSpec((1, H, D), lambda e: (e, 0, 0)),
            pl.BlockSpec((1, 1, D), lambda e: (e, 0, 0)),
            pl.BlockSpec((1, 1, K), lambda e: (e, 0, 0)),
            pl.BlockSpec((1, 1, K), lambda e: (e, 0, 0)),
            pl.BlockSpec(memory_space=pltpu.MemorySpace.HBM),
        ],
        out_specs=pl.BlockSpec((S, D), lambda e: (0, 0)),
        out_shape=jax.ShapeDtypeStruct((S, D), _F32),
        scratch_shapes=[pltpu.SemaphoreType.DMA],
        compiler_params=pltpu.CompilerParams(
            vmem_limit_bytes=65 * 1024 * 1024),
    )(ein, W1, b1, W2, b2, gating, idx, shared)


def kernel(x, gate_weight, W1, b1, W2, b2, cp_w1, cp_b1, cp_w2, cp_b2,
           sw1, sb1, sw2, sb2):
    Bx, sx, Dx = x.shape
    xf = x.reshape(S, D)

    scoresT, aux = _gating(xf, gate_weight.T)
    cp, shared = _dense(xf, cp_w1, cp_b1.reshape(1, D), cp_w2,
                        cp_b2.reshape(1, E), sw1, sb1.reshape(1, HS), sw2,
                        sb2.reshape(1, D))
    index, gating, onesT, ein = _route(scoresT, aux, xf)

    y = _moe_combine(ein.reshape(E, K, D), W1, b1.reshape(E, 1, H),
                     W2, b2.reshape(E, 1, D), gating.reshape(E, 1, K),
                     index.reshape(E, 1, K), shared)
    x_out = y.reshape(Bx, sx, Dx)
    ones = onesT.T.reshape(Bx, sx, E)
    cp_out = cp.reshape(Bx, sx, E)
    return (x_out, ones, cp_out)
